# speedup vs baseline: 1.7117x; 1.7117x over previous
import functools

import jax
import jax.numpy as jnp
from jax import lax
from jax.experimental import pallas as pl
from jax.experimental.pallas import tpu as pltpu
from jax.experimental.pallas import tpu_sc as plsc

_NUM_CORES = 2
_NUM_SUBCORES = 16
_NUM_WORKERS = _NUM_CORES * _NUM_SUBCORES
_CHUNK = 32


def _scale_table(pe, alpha2d, beta2d):
    rows, hidden = pe.shape
    blk = 1024

    def body(pe_ref, a_ref, b_ref, o_ref):
        o_ref[...] = pe_ref[...] * a_ref[...] + b_ref[...]

    return pl.pallas_call(
        body,
        grid=(rows // blk,),
        in_specs=[
            pl.BlockSpec((blk, hidden), lambda i: (i, 0)),
            pl.BlockSpec((1, hidden), lambda i: (0, 0)),
            pl.BlockSpec((1, hidden), lambda i: (0, 0)),
        ],
        out_specs=pl.BlockSpec((blk, hidden), lambda i: (i, 0)),
        out_shape=jax.ShapeDtypeStruct((rows, hidden), pe.dtype),
    )(pe, alpha2d, beta2d)


def _sc_gather(table, idx_flat):
    n_idx = idx_flat.shape[0]
    hidden = table.shape[1]
    per_worker = n_idx // _NUM_WORKERS
    mesh = plsc.VectorSubcoreMesh(core_axis_name="c", subcore_axis_name="s")

    @functools.partial(
        pl.kernel,
        out_type=jax.ShapeDtypeStruct((n_idx, hidden), table.dtype),
        mesh=mesh,
        scratch_types=[
            pltpu.VMEM((per_worker,), jnp.int32),
            pltpu.VMEM((_CHUNK, hidden), table.dtype),
            pltpu.SemaphoreType.DMA,
        ],
    )
    def kern(table_hbm, idx_hbm, out_hbm, idx_v, rows_v, sem):
        wid = lax.axis_index("s") * _NUM_CORES + lax.axis_index("c")
        base = wid * per_worker
        pltpu.sync_copy(idx_hbm.at[pl.ds(base, per_worker)], idx_v)

        @pl.loop(0, per_worker, step=_CHUNK)
        def _(c):
            pltpu.async_copy(
                table_hbm.at[idx_v.at[pl.ds(c, _CHUNK)]], rows_v, sem
            ).wait()
            pltpu.sync_copy(rows_v, out_hbm.at[pl.ds(base + c, _CHUNK)])

    return kern(table, idx_flat)


def kernel(position_ids, pe, alpha, beta):
    batch, seq = position_ids.shape
    hidden = pe.shape[1]
    scaled = _scale_table(pe, alpha.reshape(1, hidden), beta.reshape(1, hidden))
    out = _sc_gather(scaled, position_ids.reshape(batch * seq))
    return out.reshape(batch, seq, hidden)


# speedup vs baseline: 1.9962x; 1.1662x over previous
import functools

import jax
import jax.numpy as jnp
from jax import lax
from jax.experimental import pallas as pl
from jax.experimental.pallas import tpu as pltpu
from jax.experimental.pallas import tpu_sc as plsc

_NUM_CORES = 2
_NUM_SUBCORES = 16
_NUM_WORKERS = _NUM_CORES * _NUM_SUBCORES
_CHUNK = 32


def _scale_table(pe, alpha2d, beta2d):
    rows, hidden = pe.shape
    blk = 1024

    def body(pe_ref, a_ref, b_ref, o_ref):
        o_ref[...] = pe_ref[...] * a_ref[...] + b_ref[...]

    return pl.pallas_call(
        body,
        grid=(rows // blk,),
        in_specs=[
            pl.BlockSpec((blk, hidden), lambda i: (i, 0)),
            pl.BlockSpec((1, hidden), lambda i: (0, 0)),
            pl.BlockSpec((1, hidden), lambda i: (0, 0)),
        ],
        out_specs=pl.BlockSpec((blk, hidden), lambda i: (i, 0)),
        out_shape=jax.ShapeDtypeStruct((rows, hidden), pe.dtype),
    )(pe, alpha2d, beta2d)


def _sc_gather(table, idx_flat):
    n_idx = idx_flat.shape[0]
    hidden = table.shape[1]
    per_worker = n_idx // _NUM_WORKERS
    mesh = plsc.VectorSubcoreMesh(core_axis_name="c", subcore_axis_name="s")

    @functools.partial(
        pl.kernel,
        out_type=jax.ShapeDtypeStruct((n_idx, hidden), table.dtype),
        mesh=mesh,
        scratch_types=[
            pltpu.VMEM((per_worker,), jnp.int32),
            pltpu.VMEM((_CHUNK, hidden), table.dtype),
            pltpu.VMEM((_CHUNK, hidden), table.dtype),
            pltpu.SemaphoreType.DMA,
            pltpu.SemaphoreType.DMA,
            pltpu.SemaphoreType.DMA,
            pltpu.SemaphoreType.DMA,
        ],
    )
    def kern(table_hbm, idx_hbm, out_hbm, idx_v, buf0, buf1,
             sem_g0, sem_g1, sem_o0, sem_o1):
        wid = lax.axis_index("s") * _NUM_CORES + lax.axis_index("c")
        base = wid * per_worker
        pltpu.sync_copy(idx_hbm.at[pl.ds(base, per_worker)], idx_v)

        def gather(c, buf, sem):
            return pltpu.async_copy(
                table_hbm.at[idx_v.at[pl.ds(c, _CHUNK)]], buf, sem
            )

        def put(c, buf, sem):
            return pltpu.async_copy(buf, out_hbm.at[pl.ds(base + c, _CHUNK)], sem)

        gather(0, buf0, sem_g0)

        @pl.loop(0, per_worker, step=2 * _CHUNK)
        def _(c):
            @pl.when(c > 0)
            def _():
                pltpu.make_async_copy(
                    buf1, out_hbm.at[pl.ds(base + c - _CHUNK, _CHUNK)], sem_o1
                ).wait()

            gather(c + _CHUNK, buf1, sem_g1)
            pltpu.make_async_copy(
                table_hbm.at[idx_v.at[pl.ds(c, _CHUNK)]], buf0, sem_g0
            ).wait()
            put(c, buf0, sem_o0)

            @pl.when(c + 2 * _CHUNK < per_worker)
            def _():
                pltpu.make_async_copy(
                    buf0, out_hbm.at[pl.ds(base + c, _CHUNK)], sem_o0
                ).wait()
                gather(c + 2 * _CHUNK, buf0, sem_g0)

            pltpu.make_async_copy(
                table_hbm.at[idx_v.at[pl.ds(c + _CHUNK, _CHUNK)]], buf1, sem_g1
            ).wait()
            put(c + _CHUNK, buf1, sem_o1)

        pltpu.make_async_copy(
            buf0, out_hbm.at[pl.ds(base + per_worker - 2 * _CHUNK, _CHUNK)], sem_o0
        ).wait()
        pltpu.make_async_copy(
            buf1, out_hbm.at[pl.ds(base + per_worker - _CHUNK, _CHUNK)], sem_o1
        ).wait()

    return kern(table, idx_flat)


def kernel(position_ids, pe, alpha, beta):
    batch, seq = position_ids.shape
    hidden = pe.shape[1]
    scaled = _scale_table(pe, alpha.reshape(1, hidden), beta.reshape(1, hidden))
    out = _sc_gather(scaled, position_ids.reshape(batch * seq))
    return out.reshape(batch, seq, hidden)
